# baseline (speedup 1.0000x reference)
import functools

import jax
import jax.numpy as jnp
from jax import lax
from jax.experimental import pallas as pl
from jax.experimental.pallas import tpu as pltpu
from jax.experimental.pallas import tpu_sc as plsc

N_NODES = 10000
N_EDGES = 320000
H = 64
H2 = 2 * H
NUM_LAYERS = 3

NC = 2
NS = 16
NW = NC * NS
EPW = N_EDGES // NW
CH = 80
NCH = EPW // CH
GRP = 5
NG = NCH // GRP
GE = GRP * CH
ROWS_PER_TILE = N_NODES // NS

_f32 = jnp.float32

_sc_mesh = plsc.VectorSubcoreMesh(core_axis_name="c", subcore_axis_name="s")
_sc_params = pltpu.CompilerParams(use_tc_tiling_on_sc=False)


@functools.partial(
    pl.kernel,
    out_type=jax.ShapeDtypeStruct((N_EDGES, H2), _f32),
    mesh=_sc_mesh,
    scratch_types=[
        pltpu.VMEM((NCH, CH), jnp.int32),
        pltpu.VMEM((NCH, CH), jnp.int32),
        pltpu.VMEM((2, GE, H), _f32),
        pltpu.VMEM((2, GE, H), _f32),
        pltpu.SemaphoreType.DMA,
    ],
    compiler_params=_sc_params,
)
def _sc_gather(h_hbm, row_hbm, col_hbm, gcat_hbm,
               idx_r, idx_c, rbuf, cbuf, semg):
    wid = lax.axis_index("s") * NC + lax.axis_index("c")
    base = wid * EPW
    pltpu.sync_copy(row_hbm.at[wid], idx_r)
    pltpu.sync_copy(col_hbm.at[wid], idx_c)

    def fire(g, s):
        for k in range(GRP):
            ck = g * GRP + k
            pltpu.async_copy(
                h_hbm.at[idx_r.at[ck]], rbuf.at[s, pl.ds(k * CH, CH)], semg)
            pltpu.async_copy(
                h_hbm.at[idx_c.at[ck]], cbuf.at[s, pl.ds(k * CH, CH)], semg)

    def drain_write(g, s):
        for k in range(GRP):
            pltpu.make_async_copy(
                h_hbm.at[pl.ds(0, CH)], rbuf.at[s, pl.ds(k * CH, CH)],
                semg).wait()
            pltpu.make_async_copy(
                h_hbm.at[pl.ds(0, CH)], cbuf.at[s, pl.ds(k * CH, CH)],
                semg).wait()
        goff = base + g * GE
        pltpu.sync_copy(rbuf.at[s], gcat_hbm.at[pl.ds(goff, GE), pl.ds(0, H)])
        pltpu.sync_copy(cbuf.at[s], gcat_hbm.at[pl.ds(goff, GE), pl.ds(H, H)])

    fire(0, 0)

    @pl.loop(0, (NG - 1) // 2)
    def _(pg):
        g = 2 * pg
        fire(g + 1, 1)
        drain_write(g, 0)
        fire(g + 2, 0)
        drain_write(g + 1, 1)

    drain_write(NG - 1, 0)


@functools.partial(
    pl.kernel,
    out_type=jax.ShapeDtypeStruct((NC * N_NODES, H), _f32),
    mesh=_sc_mesh,
    scratch_types=[
        pltpu.VMEM((NCH, CH), jnp.int32),
        pltpu.VMEM((GE, H), _f32),
        pltpu.VMEM_SHARED((N_NODES, H), _f32),
        pltpu.SemaphoreType.DMA,
    ],
    compiler_params=_sc_params,
)
def _sc_scatter(epair_hbm, col_hbm, zeros_hbm, out_hbm, idx_c, ebuf, acc, sem):
    cid = lax.axis_index("c")
    sid = lax.axis_index("s")
    wid = sid * NC + cid
    r0 = sid * ROWS_PER_TILE
    pltpu.sync_copy(zeros_hbm.at[pl.ds(r0, ROWS_PER_TILE)],
                    acc.at[pl.ds(r0, ROWS_PER_TILE)])
    pltpu.sync_copy(col_hbm.at[wid], idx_c)
    plsc.subcore_barrier()
    base = wid * EPW

    @pl.loop(0, NG)
    def _(g):
        goff = base + g * GE
        pltpu.sync_copy(epair_hbm.at[pl.ds(goff, GE), pl.ds(0, H)], ebuf)
        cps = []
        for k in range(GRP):
            ck = g * GRP + k
            cps.append(pltpu.async_copy(
                ebuf.at[pl.ds(k * CH, CH)], acc.at[idx_c.at[ck]], sem,
                add=True))
        for cp in cps:
            cp.wait()

    plsc.subcore_barrier()
    pltpu.sync_copy(acc.at[pl.ds(r0, ROWS_PER_TILE)],
                    out_hbm.at[pl.ds(cid * N_NODES + r0, ROWS_PER_TILE)])


def _ln(t, g, b):
    mu = jnp.mean(t, axis=-1, keepdims=True)
    d = t - mu
    var = jnp.mean(d * d, axis=-1, keepdims=True)
    return d * lax.rsqrt(var + 1e-5) * g + b


def _dot(a, b):
    return jnp.dot(a, b, preferred_element_type=_f32)


def _init_body(x, W, b, hout):
    hout[...] = _dot(x[...], W[...]) + b[...]


def _edge_mlp(gcat, ev, W1rc, W1e, b1, g1, be1, W2, b2, g2, be2):
    t = _dot(gcat, W1rc[...]) + _dot(ev, W1e[...]) + b1[...]
    t = jnp.maximum(_ln(t, g1[...], be1[...]), 0.0)
    return _ln(_dot(t, W2[...]) + b2[...], g2[...], be2[...])


def _e0_body(ea, edW, edb, eout):
    eout[...] = _dot(ea[...], edW[...]) + edb[...]


def _edge0_body(gcat, e0, W1rc, W1e, b1, g1, be1,
                W2, b2, g2, be2, epair_out):
    ev = e0[...]
    u = _edge_mlp(gcat[...], ev, W1rc, W1e, b1, g1, be1, W2, b2, g2, be2)
    epair_out[...] = jnp.concatenate([u, ev + u], axis=-1)


def _edge_body(gcat, epair, W1rc, W1e, b1, g1, be1,
               W2, b2, g2, be2, epair_out):
    ev = epair[...][:, H:]
    u = _edge_mlp(gcat[...], ev, W1rc, W1e, b1, g1, be1, W2, b2, g2, be2)
    epair_out[...] = jnp.concatenate([u, ev + u], axis=-1)


def _edge_last_body(gcat, epair, W1rc, W1e, b1, g1, be1,
                    W2, b2, g2, be2, epad_out, enext_out):
    ev = epair[...][:, H:]
    u = _edge_mlp(gcat[...], ev, W1rc, W1e, b1, g1, be1, W2, b2, g2, be2)
    epad_out[...] = jnp.concatenate([u, u], axis=-1)
    enext_out[...] = ev + u


def _node_body(h, parts, W1h, W1a, b1, g1, be1, W2, b2, g2, be2, hout):
    hv = h[...]
    a = parts[0] + parts[1]
    t = _dot(hv, W1h[...]) + _dot(a, W1a[...]) + b1[...]
    t = jnp.maximum(_ln(t, g1[...], be1[...]), 0.0)
    u = _ln(_dot(t, W2[...]) + b2[...], g2[...], be2[...])
    hout[...] = hv + u


BE = 2000
BN = 2000


def _wspec(shape):
    return pl.BlockSpec(shape, lambda i, _s=len(shape): (0,) * _s)


def _tc_init(x, W, b):
    return pl.pallas_call(
        _init_body,
        grid=(N_NODES // BN,),
        in_specs=[pl.BlockSpec((BN, x.shape[1]), lambda i: (i, 0)),
                  _wspec(W.shape), _wspec(b.shape)],
        out_specs=pl.BlockSpec((BN, H), lambda i: (i, 0)),
        out_shape=jax.ShapeDtypeStruct((N_NODES, H), _f32),
    )(x, W, b)


def _tc_edge(body, arrays, weights, out_shapes):
    aspecs = [pl.BlockSpec((BE, a.shape[1]), lambda i: (i, 0)) for a in arrays]
    wspecs = [_wspec(w.shape) for w in weights]
    ospecs = tuple(pl.BlockSpec((BE, s[1]), lambda i: (i, 0))
                   for s in out_shapes)
    oshapes = tuple(jax.ShapeDtypeStruct(s, _f32) for s in out_shapes)
    if len(out_shapes) == 1:
        ospecs, oshapes = ospecs[0], oshapes[0]
    return pl.pallas_call(
        body,
        grid=(N_EDGES // BE,),
        in_specs=aspecs + wspecs,
        out_specs=ospecs,
        out_shape=oshapes,
    )(*arrays, *weights)


def _tc_node(h, parts, weights):
    return pl.pallas_call(
        _node_body,
        grid=(N_NODES // BN,),
        in_specs=[pl.BlockSpec((BN, H), lambda i: (i, 0)),
                  pl.BlockSpec((NC, BN, H), lambda i: (0, i, 0))]
                 + [_wspec(w.shape) for w in weights],
        out_specs=pl.BlockSpec((BN, H), lambda i: (i, 0)),
        out_shape=jax.ShapeDtypeStruct((N_NODES, H), _f32),
    )(h, parts, *weights)


def kernel(x, edge_index, edge_attr, params):
    p = params
    row2 = edge_index[0].reshape(NW, NCH, CH)
    col2 = edge_index[1].reshape(NW, NCH, CH)
    zeros_nodes = jnp.zeros((N_NODES, H), _f32)

    def r1(v):
        return v.reshape(1, H)

    h = _tc_init(x, p['in_W'], r1(p['in_b']))
    e0 = _tc_edge(_e0_body, [edge_attr], [p['ed_W'], r1(p['ed_b'])],
                  [(N_EDGES, H)])
    e = None
    for l in range(NUM_LAYERS):
        pe = 'l%d_e_' % l
        pn = 'l%d_n_' % l
        W1 = p[pe + 'W1']
        ew = [W1[:H2], W1[H2:], r1(p[pe + 'b1']), r1(p[pe + 'g1']),
              r1(p[pe + 'be1']), p[pe + 'W2'], r1(p[pe + 'b2']),
              r1(p[pe + 'g2']), r1(p[pe + 'be2'])]
        gcat = _sc_gather(h, row2, col2)
        if l == 0:
            epair = _tc_edge(_edge0_body, [gcat, e0], ew, [(N_EDGES, H2)])
        elif l < NUM_LAYERS - 1:
            epair = _tc_edge(_edge_body, [gcat, epair], ew, [(N_EDGES, H2)])
        else:
            epair, e = _tc_edge(_edge_last_body, [gcat, epair], ew,
                                [(N_EDGES, H2), (N_EDGES, H)])
        parts = _sc_scatter(epair, col2, zeros_nodes).reshape(NC, N_NODES, H)
        nW1 = p[pn + 'W1']
        nw = [nW1[:H], nW1[H:], r1(p[pn + 'b1']), r1(p[pn + 'g1']),
              r1(p[pn + 'be1']), p[pn + 'W2'], r1(p[pn + 'b2']),
              r1(p[pn + 'g2']), r1(p[pn + 'be2'])]
        h = _tc_node(h, parts, nw)
    return (h, e)


# speedup vs baseline: 1.0260x; 1.0260x over previous
import functools

import jax
import jax.numpy as jnp
from jax import lax
from jax.experimental import pallas as pl
from jax.experimental.pallas import tpu as pltpu
from jax.experimental.pallas import tpu_sc as plsc

N_NODES = 10000
N_EDGES = 320000
EDGE_DIM = 16
H = 64
H2 = 2 * H
NUM_LAYERS = 3

NC = 2
NS = 16
NW = NC * NS
EPW = N_EDGES // NW
CH = 100
NCH = EPW // CH
GRP = 4
NG = NCH // GRP
GE = GRP * CH
ROWS_PER_TILE = N_NODES // NS

_f32 = jnp.float32

_sc_mesh = plsc.VectorSubcoreMesh(core_axis_name="c", subcore_axis_name="s")
_sc_params = pltpu.CompilerParams(use_tc_tiling_on_sc=False)


@functools.partial(
    pl.kernel,
    out_type=jax.ShapeDtypeStruct((N_EDGES, H2), _f32),
    mesh=_sc_mesh,
    scratch_types=[
        pltpu.VMEM((NCH, CH), jnp.int32),
        pltpu.VMEM((NCH, CH), jnp.int32),
        pltpu.VMEM((2, GE, H), _f32),
        pltpu.VMEM((2, GE, H), _f32),
        pltpu.SemaphoreType.DMA,
    ],
    compiler_params=_sc_params,
)
def _sc_gather(h_hbm, row_hbm, col_hbm, gcat_hbm,
               idx_r, idx_c, rbuf, cbuf, semg):
    wid = lax.axis_index("s") * NC + lax.axis_index("c")
    base = wid * EPW
    pltpu.sync_copy(row_hbm.at[wid], idx_r)
    pltpu.sync_copy(col_hbm.at[wid], idx_c)

    def fire(g, s):
        for k in range(GRP):
            ck = g * GRP + k
            pltpu.async_copy(
                h_hbm.at[idx_r.at[ck]], rbuf.at[s, pl.ds(k * CH, CH)], semg)
            pltpu.async_copy(
                h_hbm.at[idx_c.at[ck]], cbuf.at[s, pl.ds(k * CH, CH)], semg)

    def drain_write(g, s):
        for k in range(GRP):
            pltpu.make_async_copy(
                h_hbm.at[pl.ds(0, CH)], rbuf.at[s, pl.ds(k * CH, CH)],
                semg).wait()
            pltpu.make_async_copy(
                h_hbm.at[pl.ds(0, CH)], cbuf.at[s, pl.ds(k * CH, CH)],
                semg).wait()
        goff = base + g * GE
        pltpu.sync_copy(rbuf.at[s], gcat_hbm.at[pl.ds(goff, GE), pl.ds(0, H)])
        pltpu.sync_copy(cbuf.at[s], gcat_hbm.at[pl.ds(goff, GE), pl.ds(H, H)])

    fire(0, 0)

    @pl.loop(0, (NG - 1) // 2)
    def _(pg):
        g = 2 * pg
        fire(g + 1, 1)
        drain_write(g, 0)
        fire(g + 2, 0)
        drain_write(g + 1, 1)

    drain_write(NG - 1, 0)


@functools.partial(
    pl.kernel,
    out_type=jax.ShapeDtypeStruct((NC * N_NODES, H), _f32),
    mesh=_sc_mesh,
    scratch_types=[
        pltpu.VMEM((NCH, CH), jnp.int32),
        pltpu.VMEM((GE, H), _f32),
        pltpu.VMEM_SHARED((N_NODES, H), _f32),
        pltpu.SemaphoreType.DMA,
    ],
    compiler_params=_sc_params,
)
def _sc_scatter(epair_hbm, col_hbm, zeros_hbm, out_hbm, idx_c, ebuf, acc, sem):
    cid = lax.axis_index("c")
    sid = lax.axis_index("s")
    wid = sid * NC + cid
    r0 = sid * ROWS_PER_TILE
    pltpu.sync_copy(zeros_hbm.at[pl.ds(r0, ROWS_PER_TILE)],
                    acc.at[pl.ds(r0, ROWS_PER_TILE)])
    pltpu.sync_copy(col_hbm.at[wid], idx_c)
    plsc.subcore_barrier()
    base = wid * EPW

    @pl.loop(0, NG)
    def _(g):
        goff = base + g * GE
        pltpu.sync_copy(epair_hbm.at[pl.ds(goff, GE), pl.ds(0, H)], ebuf)
        cps = []
        for k in range(GRP):
            ck = g * GRP + k
            cps.append(pltpu.async_copy(
                ebuf.at[pl.ds(k * CH, CH)], acc.at[idx_c.at[ck]], sem,
                add=True))
        for cp in cps:
            cp.wait()

    plsc.subcore_barrier()
    pltpu.sync_copy(acc.at[pl.ds(r0, ROWS_PER_TILE)],
                    out_hbm.at[pl.ds(cid * N_NODES + r0, ROWS_PER_TILE)])


def _ln(t, g, b):
    mu = jnp.mean(t, axis=-1, keepdims=True)
    d = t - mu
    var = jnp.mean(d * d, axis=-1, keepdims=True)
    return d * lax.rsqrt(var + 1e-5) * g + b


def _dot(a, b):
    return jnp.dot(a, b, preferred_element_type=_f32)


def _init_body(x, W, b, hout):
    hout[...] = _dot(x[...], W[...]) + b[...]


def _edge_mlp(gcat, ev, W1rc, W1e, b1, g1, be1, W2, b2, g2, be2):
    t = _dot(gcat, W1rc[...]) + _dot(ev, W1e[...]) + b1[...]
    t = jnp.maximum(_ln(t, g1[...], be1[...]), 0.0)
    return _ln(_dot(t, W2[...]) + b2[...], g2[...], be2[...])


def _edge0_body(gcat, ea, edW, edb, W1rc, W1e, b1, g1, be1,
                W2, b2, g2, be2, epair_out):
    ev = _dot(ea[...], edW[...]) + edb[...]
    u = _edge_mlp(gcat[...], ev, W1rc, W1e, b1, g1, be1, W2, b2, g2, be2)
    epair_out[...] = jnp.concatenate([u, ev + u], axis=-1)


def _edge_body(gcat, epair, W1rc, W1e, b1, g1, be1,
               W2, b2, g2, be2, epair_out):
    ev = epair[...][:, H:]
    u = _edge_mlp(gcat[...], ev, W1rc, W1e, b1, g1, be1, W2, b2, g2, be2)
    epair_out[...] = jnp.concatenate([u, ev + u], axis=-1)


def _edge_last_body(gcat, epair, W1rc, W1e, b1, g1, be1,
                    W2, b2, g2, be2, epad_out, enext_out):
    ev = epair[...][:, H:]
    u = _edge_mlp(gcat[...], ev, W1rc, W1e, b1, g1, be1, W2, b2, g2, be2)
    epad_out[...] = jnp.concatenate([u, u], axis=-1)
    enext_out[...] = ev + u


def _node_body(h, parts, W1h, W1a, b1, g1, be1, W2, b2, g2, be2, hout):
    hv = h[...]
    a = parts[0] + parts[1]
    t = _dot(hv, W1h[...]) + _dot(a, W1a[...]) + b1[...]
    t = jnp.maximum(_ln(t, g1[...], be1[...]), 0.0)
    u = _ln(_dot(t, W2[...]) + b2[...], g2[...], be2[...])
    hout[...] = hv + u


BE = 2000
BN = 2000


def _wspec(shape):
    return pl.BlockSpec(shape, lambda i, _s=len(shape): (0,) * _s)


def _tc_init(x, W, b):
    return pl.pallas_call(
        _init_body,
        grid=(N_NODES // BN,),
        in_specs=[pl.BlockSpec((BN, x.shape[1]), lambda i: (i, 0)),
                  _wspec(W.shape), _wspec(b.shape)],
        out_specs=pl.BlockSpec((BN, H), lambda i: (i, 0)),
        out_shape=jax.ShapeDtypeStruct((N_NODES, H), _f32),
    )(x, W, b)


NEB = N_EDGES // 2000


def _tc_edge(body, arrays, weights, out_shapes):
    aspecs = [pl.BlockSpec((a.shape[0] // NEB, a.shape[1]), lambda i: (i, 0))
              for a in arrays]
    wspecs = [_wspec(w.shape) for w in weights]
    ospecs = tuple(pl.BlockSpec((s[0] // NEB, s[1]), lambda i: (i, 0))
                   for s in out_shapes)
    oshapes = tuple(jax.ShapeDtypeStruct(s, _f32) for s in out_shapes)
    if len(out_shapes) == 1:
        ospecs, oshapes = ospecs[0], oshapes[0]
    return pl.pallas_call(
        body,
        grid=(NEB,),
        in_specs=aspecs + wspecs,
        out_specs=ospecs,
        out_shape=oshapes,
    )(*arrays, *weights)


def _tc_node(h, parts, weights):
    return pl.pallas_call(
        _node_body,
        grid=(N_NODES // BN,),
        in_specs=[pl.BlockSpec((BN, H), lambda i: (i, 0)),
                  pl.BlockSpec((NC, BN, H), lambda i: (0, i, 0))]
                 + [_wspec(w.shape) for w in weights],
        out_specs=pl.BlockSpec((BN, H), lambda i: (i, 0)),
        out_shape=jax.ShapeDtypeStruct((N_NODES, H), _f32),
    )(h, parts, *weights)


def kernel(x, edge_index, edge_attr, params):
    p = params
    row2 = edge_index[0].reshape(NW, NCH, CH)
    col2 = edge_index[1].reshape(NW, NCH, CH)
    zeros_nodes = jnp.zeros((N_NODES, H), _f32)

    def r1(v):
        return v.reshape(1, H)

    h = _tc_init(x, p['in_W'], r1(p['in_b']))
    e = None
    for l in range(NUM_LAYERS):
        pe = 'l%d_e_' % l
        pn = 'l%d_n_' % l
        W1 = p[pe + 'W1']
        ew = [W1[:H2], W1[H2:], r1(p[pe + 'b1']), r1(p[pe + 'g1']),
              r1(p[pe + 'be1']), p[pe + 'W2'], r1(p[pe + 'b2']),
              r1(p[pe + 'g2']), r1(p[pe + 'be2'])]
        gcat = _sc_gather(h, row2, col2)
        if l == 0:
            epair = _tc_edge(
                _edge0_body, [gcat, edge_attr],
                [p['ed_W'], r1(p['ed_b'])] + ew, [(N_EDGES, H2)])
        elif l < NUM_LAYERS - 1:
            epair = _tc_edge(_edge_body, [gcat, epair], ew, [(N_EDGES, H2)])
        else:
            epair, e = _tc_edge(_edge_last_body, [gcat, epair], ew,
                                [(N_EDGES, H2), (N_EDGES, H)])
        parts = _sc_scatter(epair, col2, zeros_nodes).reshape(NC, N_NODES, H)
        nW1 = p[pn + 'W1']
        nw = [nW1[:H], nW1[H:], r1(p[pn + 'b1']), r1(p[pn + 'g1']),
              r1(p[pn + 'be1']), p[pn + 'W2'], r1(p[pn + 'b2']),
              r1(p[pn + 'g2']), r1(p[pn + 'be2'])]
        h = _tc_node(h, parts, nw)
    return (h, e)


# speedup vs baseline: 1.0535x; 1.0267x over previous
import functools

import jax
import jax.numpy as jnp
from jax import lax
from jax.experimental import pallas as pl
from jax.experimental.pallas import tpu as pltpu
from jax.experimental.pallas import tpu_sc as plsc

N_NODES = 10000
N_EDGES = 320000
EDGE_DIM = 16
H = 64
H2 = 2 * H
NUM_LAYERS = 3

NC = 2
NS = 16
NW = NC * NS
NSPLIT = 2
EH = N_EDGES // NSPLIT
EPW = EH // NW
CH = 100
NCH = EPW // CH
GRP = 2
NG = NCH // GRP
GE = GRP * CH
SGRP = 5
SNG = NCH // SGRP
SGE = SGRP * CH
ROWS_PER_TILE = N_NODES // NS

_f32 = jnp.float32

_sc_mesh = plsc.VectorSubcoreMesh(core_axis_name="c", subcore_axis_name="s")
_sc_params = pltpu.CompilerParams(use_tc_tiling_on_sc=False)


@functools.partial(
    pl.kernel,
    out_type=jax.ShapeDtypeStruct((EH, H2), _f32),
    mesh=_sc_mesh,
    scratch_types=[
        pltpu.VMEM((NCH, CH), jnp.int32),
        pltpu.VMEM((NCH, CH), jnp.int32),
        pltpu.VMEM((2, GE, H), _f32),
        pltpu.VMEM((2, GE, H), _f32),
        pltpu.SemaphoreType.DMA,
    ],
    compiler_params=_sc_params,
)
def _sc_gather(h_hbm, row_hbm, col_hbm, gcat_hbm,
               idx_r, idx_c, rbuf, cbuf, semg):
    wid = lax.axis_index("s") * NC + lax.axis_index("c")
    base = wid * EPW
    pltpu.sync_copy(row_hbm.at[wid], idx_r)
    pltpu.sync_copy(col_hbm.at[wid], idx_c)

    def fire(g, s):
        for k in range(GRP):
            ck = g * GRP + k
            pltpu.async_copy(
                h_hbm.at[idx_r.at[ck]], rbuf.at[s, pl.ds(k * CH, CH)], semg)
            pltpu.async_copy(
                h_hbm.at[idx_c.at[ck]], cbuf.at[s, pl.ds(k * CH, CH)], semg)

    def drain_write(g, s):
        for k in range(GRP):
            pltpu.make_async_copy(
                h_hbm.at[pl.ds(0, CH)], rbuf.at[s, pl.ds(k * CH, CH)],
                semg).wait()
            pltpu.make_async_copy(
                h_hbm.at[pl.ds(0, CH)], cbuf.at[s, pl.ds(k * CH, CH)],
                semg).wait()
        goff = base + g * GE
        pltpu.sync_copy(rbuf.at[s], gcat_hbm.at[pl.ds(goff, GE), pl.ds(0, H)])
        pltpu.sync_copy(cbuf.at[s], gcat_hbm.at[pl.ds(goff, GE), pl.ds(H, H)])

    fire(0, 0)

    @pl.loop(0, (NG - 1) // 2)
    def _(pg):
        g = 2 * pg
        fire(g + 1, 1)
        drain_write(g, 0)
        fire(g + 2, 0)
        drain_write(g + 1, 1)

    drain_write(NG - 1, 0)


@functools.partial(
    pl.kernel,
    out_type=jax.ShapeDtypeStruct((NC * N_NODES, H), _f32),
    mesh=_sc_mesh,
    scratch_types=[
        pltpu.VMEM((NCH, CH), jnp.int32),
        pltpu.VMEM((SGE, H), _f32),
        pltpu.VMEM_SHARED((N_NODES, H), _f32),
        pltpu.SemaphoreType.DMA,
    ],
    compiler_params=_sc_params,
)
def _sc_scatter(epair_hbm, col_hbm, zeros_hbm, out_hbm, idx_c, ebuf, acc, sem):
    cid = lax.axis_index("c")
    sid = lax.axis_index("s")
    wid = sid * NC + cid
    r0 = sid * ROWS_PER_TILE
    pltpu.sync_copy(zeros_hbm.at[pl.ds(r0, ROWS_PER_TILE)],
                    acc.at[pl.ds(r0, ROWS_PER_TILE)])
    pltpu.sync_copy(col_hbm.at[wid], idx_c)
    plsc.subcore_barrier()
    base = wid * EPW

    @pl.loop(0, SNG)
    def _(g):
        goff = base + g * SGE
        pltpu.sync_copy(epair_hbm.at[pl.ds(goff, SGE), pl.ds(0, H)], ebuf)
        cps = []
        for k in range(SGRP):
            ck = g * SGRP + k
            cps.append(pltpu.async_copy(
                ebuf.at[pl.ds(k * CH, CH)], acc.at[idx_c.at[ck]], sem,
                add=True))
        for cp in cps:
            cp.wait()

    plsc.subcore_barrier()
    pltpu.sync_copy(acc.at[pl.ds(r0, ROWS_PER_TILE)],
                    out_hbm.at[pl.ds(cid * N_NODES + r0, ROWS_PER_TILE)])


def _ln(t, g, b):
    mu = jnp.mean(t, axis=-1, keepdims=True)
    d = t - mu
    var = jnp.mean(d * d, axis=-1, keepdims=True)
    return d * lax.rsqrt(var + 1e-5) * g + b


def _dot(a, b):
    return jnp.dot(a, b, preferred_element_type=_f32)


def _init_body(x, W, b, hout):
    hout[...] = _dot(x[...], W[...]) + b[...]


def _edge_mlp(gcat, ev, W1rc, W1e, b1, g1, be1, W2, b2, g2, be2):
    t = _dot(gcat, W1rc[...]) + _dot(ev, W1e[...]) + b1[...]
    t = jnp.maximum(_ln(t, g1[...], be1[...]), 0.0)
    return _ln(_dot(t, W2[...]) + b2[...], g2[...], be2[...])


def _edge0_body(gcat, ea, edW, edb, W1rc, W1e, b1, g1, be1,
                W2, b2, g2, be2, epair_out):
    ev = _dot(ea[...], edW[...]) + edb[...]
    u = _edge_mlp(gcat[...], ev, W1rc, W1e, b1, g1, be1, W2, b2, g2, be2)
    epair_out[...] = jnp.concatenate([u, ev + u], axis=-1)


def _edge_body(gcat, epair, W1rc, W1e, b1, g1, be1,
               W2, b2, g2, be2, epair_out):
    ev = epair[...][:, H:]
    u = _edge_mlp(gcat[...], ev, W1rc, W1e, b1, g1, be1, W2, b2, g2, be2)
    epair_out[...] = jnp.concatenate([u, ev + u], axis=-1)


def _edge_last_body(gcat, epair, W1rc, W1e, b1, g1, be1,
                    W2, b2, g2, be2, epad_out, enext_out):
    ev = epair[...][:, H:]
    u = _edge_mlp(gcat[...], ev, W1rc, W1e, b1, g1, be1, W2, b2, g2, be2)
    epad_out[...] = jnp.concatenate([u, u], axis=-1)
    enext_out[...] = ev + u


def _node_body(h, pa, pb, W1h, W1a, b1, g1, be1, W2, b2, g2, be2, hout):
    hv = h[...]
    a = (pa[0] + pa[1]) + (pb[0] + pb[1])
    t = _dot(hv, W1h[...]) + _dot(a, W1a[...]) + b1[...]
    t = jnp.maximum(_ln(t, g1[...], be1[...]), 0.0)
    u = _ln(_dot(t, W2[...]) + b2[...], g2[...], be2[...])
    hout[...] = hv + u


BE = 2000
BN = 2000


def _wspec(shape):
    return pl.BlockSpec(shape, lambda i, _s=len(shape): (0,) * _s)


def _tc_init(x, W, b):
    return pl.pallas_call(
        _init_body,
        grid=(N_NODES // BN,),
        in_specs=[pl.BlockSpec((BN, x.shape[1]), lambda i: (i, 0)),
                  _wspec(W.shape), _wspec(b.shape)],
        out_specs=pl.BlockSpec((BN, H), lambda i: (i, 0)),
        out_shape=jax.ShapeDtypeStruct((N_NODES, H), _f32),
    )(x, W, b)


NEB = EH // BE


def _tc_edge(body, hf, arrays, weights, out_shapes):
    def spec(a):
        if a.shape[0] == EH:
            return pl.BlockSpec((BE, a.shape[1]), lambda i: (i, 0))
        return pl.BlockSpec((BE, a.shape[1]),
                            lambda i, _o=hf * NEB: (i + _o, 0))
    aspecs = [spec(a) for a in arrays]
    wspecs = [_wspec(w.shape) for w in weights]
    ospecs = tuple(pl.BlockSpec((BE, s[1]), lambda i: (i, 0))
                   for s in out_shapes)
    oshapes = tuple(jax.ShapeDtypeStruct(s, _f32) for s in out_shapes)
    if len(out_shapes) == 1:
        ospecs, oshapes = ospecs[0], oshapes[0]
    return pl.pallas_call(
        body,
        grid=(NEB,),
        in_specs=aspecs + wspecs,
        out_specs=ospecs,
        out_shape=oshapes,
    )(*arrays, *weights)


def _tc_node(h, parts_a, parts_b, weights):
    pspec = pl.BlockSpec((NC, BN, H), lambda i: (0, i, 0))
    return pl.pallas_call(
        _node_body,
        grid=(N_NODES // BN,),
        in_specs=[pl.BlockSpec((BN, H), lambda i: (i, 0)), pspec, pspec]
                 + [_wspec(w.shape) for w in weights],
        out_specs=pl.BlockSpec((BN, H), lambda i: (i, 0)),
        out_shape=jax.ShapeDtypeStruct((N_NODES, H), _f32),
    )(h, parts_a, parts_b, *weights)


def kernel(x, edge_index, edge_attr, params):
    p = params
    row2 = [edge_index[0, hf * EH:(hf + 1) * EH].reshape(NW, NCH, CH)
            for hf in range(NSPLIT)]
    col2 = [edge_index[1, hf * EH:(hf + 1) * EH].reshape(NW, NCH, CH)
            for hf in range(NSPLIT)]
    zeros_nodes = jnp.zeros((N_NODES, H), _f32)

    def r1(v):
        return v.reshape(1, H)

    h = _tc_init(x, p['in_W'], r1(p['in_b']))
    epair = [None] * NSPLIT
    eout = [None] * NSPLIT
    for l in range(NUM_LAYERS):
        pe = 'l%d_e_' % l
        pn = 'l%d_n_' % l
        W1 = p[pe + 'W1']
        ew = [W1[:H2], W1[H2:], r1(p[pe + 'b1']), r1(p[pe + 'g1']),
              r1(p[pe + 'be1']), p[pe + 'W2'], r1(p[pe + 'b2']),
              r1(p[pe + 'g2']), r1(p[pe + 'be2'])]
        gcat = [_sc_gather(h, row2[hf], col2[hf]) for hf in range(NSPLIT)]
        parts = [None] * NSPLIT
        for hf in range(NSPLIT):
            if l == 0:
                epair[hf] = _tc_edge(
                    _edge0_body, hf, [gcat[hf], edge_attr],
                    [p['ed_W'], r1(p['ed_b'])] + ew, [(EH, H2)])
            elif l < NUM_LAYERS - 1:
                epair[hf] = _tc_edge(_edge_body, hf, [gcat[hf], epair[hf]],
                                     ew, [(EH, H2)])
            else:
                epair[hf], eout[hf] = _tc_edge(
                    _edge_last_body, hf, [gcat[hf], epair[hf]], ew,
                    [(EH, H2), (EH, H)])
            parts[hf] = _sc_scatter(epair[hf], col2[hf],
                                    zeros_nodes).reshape(NC, N_NODES, H)
        nW1 = p[pn + 'W1']
        nw = [nW1[:H], nW1[H:], r1(p[pn + 'b1']), r1(p[pn + 'g1']),
              r1(p[pn + 'be1']), p[pn + 'W2'], r1(p[pn + 'b2']),
              r1(p[pn + 'g2']), r1(p[pn + 'be2'])]
        h = _tc_node(h, parts[0], parts[1], nw)
    e = jnp.concatenate(eout, axis=0)
    return (h, e)
